# baseline (speedup 1.0000x reference)
import functools

import jax
import jax.numpy as jnp
from jax import lax
from jax.experimental import pallas as pl
from jax.experimental.pallas import tpu as pltpu
from jax.experimental.pallas import tpu_sc as plsc

V, D, H, WD = 6, 48, 16, 44
C, OUT = 256, 256
PIX = V * H * WD
PV = H * WD
GX = GY = 128
G = GX * GY
NCHUNK = 2
CK = OUT // NCHUNK
NS = 16
PPT = PIX // NS
RPT = G // NS
LANES = 16
PB = 2
NSLOT = 2
GRP = PB * NSLOT
SROWS = PB * D
BINIT = 32
FB = 16


def _prep_body(feat_ref, w_ref, depth_ref, x_ref, y_ref,
               fw_ref, wt_ref, rk_ref):
    fw3 = lax.dot_general(feat_ref[0], w_ref[...],
                          dimension_numbers=(((0,), (0,)), ((), ())),
                          preferred_element_type=jnp.float32)
    d = depth_ref[0]
    m = jnp.max(d, axis=0, keepdims=True)
    e = jnp.exp(d - m)
    sm = e / jnp.sum(e, axis=0, keepdims=True)
    cx = jnp.floor(((x_ref[0] + 1.0) / 2.0) * 128.0)
    cy = jnp.floor(((y_ref[0] + 1.0) / 2.0) * 128.0)
    kept = (cx >= 0.0) & (cx < 128.0) & (cy >= 0.0) & (cy < 128.0)
    rank = cy.astype(jnp.int32) * GX + cx.astype(jnp.int32)
    rk3 = jnp.where(kept, rank, 0)
    wt3 = jnp.where(kept, sm, 0.0)
    for h in range(H):
        fw_ref[0, pl.ds(h * WD, WD), :] = fw3[h][:, :CK]
        fw_ref[1, pl.ds(h * WD, WD), :] = fw3[h][:, CK:]
        wt_ref[pl.ds(h * WD, WD), :] = wt3[:, h, :].T
        rk_ref[pl.ds(h * WD, WD), :] = rk3[:, h, :].T


def _sc_body(fw_hbm, wpt_hbm, rpt_hbm, bias_hbm, out_hbm,
             acc, wloc, rloc, floc, buf, binit, bvec, flb, fob, sem0, sem1):
    c = lax.axis_index("c")
    s = lax.axis_index("s")
    p0 = s * PPT
    r0 = s * RPT
    sems = (sem0, sem1)
    cc = c
    pltpu.sync_copy(bias_hbm.at[cc], bvec)

    def initbody(i, _):
        for q in range(CK // 32):
            pe = bvec[pl.ds(32 * q, LANES)]
            po = bvec[pl.ds(32 * q + LANES, LANES)]
            binit[i, pl.ds(32 * q, 32)] = plsc.pack(
                pe, po, format=plsc.PackFormat.INTERLEAVED)
        return 0
    lax.fori_loop(0, BINIT, initbody, 0)
    for j in range(RPT // BINIT):
        pltpu.sync_copy(binit, acc.at[pl.ds(r0 + j * BINIT, BINIT)])
    pltpu.sync_copy(wpt_hbm.at[pl.ds(p0, PPT)], wloc)
    pltpu.sync_copy(rpt_hbm.at[pl.ds(p0 // PB, PPT // PB)], rloc)
    plsc.subcore_barrier()

    def build(slot, grp_local, woff):
        for j in range(PB):
            pix = grp_local * PB + j
            re_ = [floc[pix, pl.ds(32 * q, LANES)] for q in range(CK // 32)]
            ro_ = [floc[pix, pl.ds(32 * q + LANES, LANES)]
                   for q in range(CK // 32)]
            wvs = [wloc[woff + pix, pl.ds(LANES * t, LANES)]
                   for t in range(D // LANES)]
            for b in range(D):
                wsc = wvs[b // LANES][b % LANES]
                for q in range(CK // 32):
                    buf[slot, j * D + b, pl.ds(32 * q, 32)] = plsc.pack(
                        re_[q] * wsc, ro_[q] * wsc,
                        format=plsc.PackFormat.INTERLEAVED)

    def fire(slot, grp, sem):
        pltpu.async_copy(buf.at[slot], acc.at[rloc.at[grp]], sem, add=True)

    def drain(slot, sem):
        pltpu.make_async_copy(buf.at[slot], acc.at[rloc.at[0]], sem).wait()

    PH = PPT // 2
    for half in range(2):
        pltpu.sync_copy(fw_hbm.at[cc, pl.ds(p0 + half * PH, PH)], floc)

        def pixbody(g, _):
            for slot in range(NSLOT):
                @pl.when(g > 0)
                def _():
                    drain(slot, sems[slot])
                gl = g * NSLOT + slot
                build(slot, gl, half * PH)
                fire(slot, half * (PH // PB) + gl, sems[slot])
            return 0
        lax.fori_loop(0, PH // GRP, pixbody, 0)
        for slot in range(NSLOT):
            drain(slot, sems[slot])
    plsc.subcore_barrier()

    NBLK = RPT // FB

    def fl_in(blk):
        pltpu.async_copy(acc.at[pl.ds(r0 + blk * FB, FB)],
                         flb.at[blk % 2], sem0)

    def fl_in_wait():
        pltpu.make_async_copy(acc.at[pl.ds(r0, FB)], flb.at[0], sem0).wait()

    def fl_out(blk):
        pltpu.async_copy(fob.at[blk % 2],
                         out_hbm.at[pl.ds(r0 + blk * FB, FB),
                                    pl.ds(cc * CK, CK)], sem1)

    def fl_out_wait():
        pltpu.make_async_copy(fob.at[0],
                              out_hbm.at[pl.ds(r0, FB), pl.ds(cc * CK, CK)],
                              sem1).wait()

    fl_in(0)
    fl_in(1)
    for blk in range(NBLK):
        sl = blk % 2
        fl_in_wait()
        if blk >= 2:
            fl_out_wait()

        def convbody(i, _):
            for q in range(CK // 32):
                a, b = plsc.unpack(flb[sl, i, pl.ds(32 * q, 32)],
                                   format=plsc.PackFormat.INTERLEAVED)
                fob[sl, i, pl.ds(32 * q, LANES)] = a
                fob[sl, i, pl.ds(32 * q + LANES, LANES)] = b
            return 0
        lax.fori_loop(0, FB, convbody, 0)
        fl_out(blk)
        if blk + 2 < NBLK:
            fl_in(blk + 2)
    fl_out_wait()
    fl_out_wait()


@functools.cache
def _sc_scatter():
    return functools.partial(
        pl.kernel,
        out_type=jax.ShapeDtypeStruct((G, OUT), jnp.float32),
        mesh=plsc.VectorSubcoreMesh(core_axis_name="c", subcore_axis_name="s"),
        compiler_params=pltpu.CompilerParams(use_tc_tiling_on_sc=False,
                                             needs_layout_passes=False),
        scratch_types=[
            pltpu.VMEM_SHARED((G, CK), jnp.bfloat16),
            pltpu.VMEM((PPT, D), jnp.float32),
            pltpu.VMEM((PPT // PB, PB * D), jnp.int32),
            pltpu.VMEM((PPT // 2, CK), jnp.float32),
            pltpu.VMEM((NSLOT, SROWS, CK), jnp.bfloat16),
            pltpu.VMEM((BINIT, CK), jnp.bfloat16),
            pltpu.VMEM((CK,), jnp.float32),
            pltpu.VMEM((2, FB, CK), jnp.bfloat16),
            pltpu.VMEM((2, FB, CK), jnp.float32),
            pltpu.SemaphoreType.DMA,
            pltpu.SemaphoreType.DMA,
        ],
    )(_sc_body)


def kernel(bev_features, frustum_features, frustum_bev_coordinates,
           last_step_depth_features, W, bias):
    feat_n = frustum_features.reshape(V, C, H, WD)
    depth_n = last_step_depth_features.reshape(V, D, H, WD)
    x_n = frustum_bev_coordinates[..., 0].reshape(V, D, H, WD)
    y_n = frustum_bev_coordinates[..., 1].reshape(V, D, H, WD)

    fw_c, wpt, rpt = pl.pallas_call(
        _prep_body,
        grid=(V,),
        in_specs=[
            pl.BlockSpec((1, C, H, WD), lambda v: (v, 0, 0, 0)),
            pl.BlockSpec((C, C), lambda v: (0, 0)),
            pl.BlockSpec((1, D, H, WD), lambda v: (v, 0, 0, 0)),
            pl.BlockSpec((1, D, H, WD), lambda v: (v, 0, 0, 0)),
            pl.BlockSpec((1, D, H, WD), lambda v: (v, 0, 0, 0)),
        ],
        out_specs=[
            pl.BlockSpec((NCHUNK, PV, CK), lambda v: (0, v, 0)),
            pl.BlockSpec((PV, D), lambda v: (v, 0)),
            pl.BlockSpec((PV, D), lambda v: (v, 0)),
        ],
        out_shape=[
            jax.ShapeDtypeStruct((NCHUNK, PIX, CK), jnp.float32),
            jax.ShapeDtypeStruct((PIX, D), jnp.float32),
            jax.ShapeDtypeStruct((PIX, D), jnp.int32),
        ],
    )(feat_n, W, depth_n, x_n, y_n)

    bias_c = bias.reshape(NCHUNK, CK)
    rpt2 = rpt.reshape(PIX // PB, PB * D)

    out = _sc_scatter()(fw_c, wpt, rpt2, bias_c)
    return out.reshape(1, G, OUT)


# speedup vs baseline: 1.0994x; 1.0994x over previous
import functools

import jax
import jax.numpy as jnp
from jax import lax
from jax.experimental import pallas as pl
from jax.experimental.pallas import tpu as pltpu
from jax.experimental.pallas import tpu_sc as plsc

V, D, H, WD = 6, 48, 16, 44
C, OUT = 256, 256
PIX = V * H * WD
PV = H * WD
GX = GY = 128
G = GX * GY
NCHUNK = 2
CK = OUT // NCHUNK
NS = 16
PPT = PIX // NS
RPT = G // NS
LANES = 16
PB = 2
NSLOT = 2
GRP = PB * NSLOT
SROWS = PB * D
BINIT = 32
FB = 16


def _prep_body(feat_ref, w_ref, depth_ref, x_ref, y_ref,
               fw_ref, wt_ref, rk_ref):
    fw = lax.dot_general(feat_ref[0], w_ref[...],
                         dimension_numbers=(((0,), (0,)), ((), ())),
                         preferred_element_type=jnp.float32)
    fw_ref[0] = fw[:, :CK]
    fw_ref[1] = fw[:, CK:]
    d = depth_ref[0]
    m = jnp.max(d, axis=0, keepdims=True)
    e = jnp.exp(d - m)
    sm = e / jnp.sum(e, axis=0, keepdims=True)
    cx = jnp.floor(((x_ref[0] + 1.0) / 2.0) * 128.0)
    cy = jnp.floor(((y_ref[0] + 1.0) / 2.0) * 128.0)
    kept = (cx >= 0.0) & (cx < 128.0) & (cy >= 0.0) & (cy < 128.0)
    rank = cy.astype(jnp.int32) * GX + cx.astype(jnp.int32)
    rk_ref[...] = jnp.where(kept, rank, 0).T
    wt_ref[...] = jnp.where(kept, sm, 0.0).T


def _sc_body(fw_hbm, wpt_hbm, rpt_hbm, bias_hbm, out_hbm,
             acc, wloc, rloc, floc, buf, binit, bvec, flb, fob, sem0, sem1):
    c = lax.axis_index("c")
    s = lax.axis_index("s")
    p0 = s * PPT
    r0 = s * RPT
    sems = (sem0, sem1)
    cc = c
    pltpu.sync_copy(bias_hbm.at[cc], bvec)

    def initbody(i, _):
        for q in range(CK // 32):
            pe = bvec[pl.ds(32 * q, LANES)]
            po = bvec[pl.ds(32 * q + LANES, LANES)]
            binit[i, pl.ds(32 * q, 32)] = plsc.pack(
                pe, po, format=plsc.PackFormat.INTERLEAVED)
        return 0
    lax.fori_loop(0, BINIT, initbody, 0)
    for j in range(RPT // BINIT):
        pltpu.async_copy(binit, acc.at[pl.ds(r0 + j * BINIT, BINIT)], sem0)
    pltpu.async_copy(wpt_hbm.at[pl.ds(p0, PPT)], wloc, sem1)
    pltpu.async_copy(rpt_hbm.at[pl.ds(p0 // PB, PPT // PB)], rloc, sem1)
    for j in range(RPT // BINIT):
        pltpu.make_async_copy(binit, acc.at[pl.ds(r0, BINIT)], sem0).wait()
    pltpu.make_async_copy(wpt_hbm.at[pl.ds(p0, PPT)], wloc, sem1).wait()
    pltpu.make_async_copy(rpt_hbm.at[pl.ds(p0 // PB, PPT // PB)], rloc,
                          sem1).wait()
    plsc.subcore_barrier()

    def build(slot, grp_local, woff):
        for j in range(PB):
            pix = grp_local * PB + j
            re_ = [floc[pix, pl.ds(32 * q, LANES)] for q in range(CK // 32)]
            ro_ = [floc[pix, pl.ds(32 * q + LANES, LANES)]
                   for q in range(CK // 32)]
            wvs = [wloc[woff + pix, pl.ds(LANES * t, LANES)]
                   for t in range(D // LANES)]
            for b in range(D):
                wsc = wvs[b // LANES][b % LANES]
                for q in range(CK // 32):
                    buf[slot, j * D + b, pl.ds(32 * q, 32)] = plsc.pack(
                        re_[q] * wsc, ro_[q] * wsc,
                        format=plsc.PackFormat.INTERLEAVED)

    def fire(slot, grp, sem):
        pltpu.async_copy(buf.at[slot], acc.at[rloc.at[grp]], sem, add=True)

    def drain(slot, sem):
        pltpu.make_async_copy(buf.at[slot], acc.at[rloc.at[0]], sem).wait()

    PH = PPT // 2
    for half in range(2):
        pltpu.sync_copy(fw_hbm.at[cc, pl.ds(p0 + half * PH, PH)], floc)

        def pixbody(g, _):
            for slot in range(NSLOT):
                @pl.when(g > 0)
                def _():
                    drain(slot, sems[slot])
                gl = g * NSLOT + slot
                build(slot, gl, half * PH)
                fire(slot, half * (PH // PB) + gl, sems[slot])
            return 0
        lax.fori_loop(0, PH // GRP, pixbody, 0)
        for slot in range(NSLOT):
            drain(slot, sems[slot])
    plsc.subcore_barrier()

    NBLK = RPT // FB

    def fl_in(blk):
        pltpu.async_copy(acc.at[pl.ds(r0 + blk * FB, FB)],
                         flb.at[blk % 2], sem0)

    def fl_in_wait():
        pltpu.make_async_copy(acc.at[pl.ds(r0, FB)], flb.at[0], sem0).wait()

    def fl_out(blk):
        pltpu.async_copy(fob.at[blk % 2],
                         out_hbm.at[pl.ds(r0 + blk * FB, FB),
                                    pl.ds(cc * CK, CK)], sem1)

    def fl_out_wait():
        pltpu.make_async_copy(fob.at[0],
                              out_hbm.at[pl.ds(r0, FB), pl.ds(cc * CK, CK)],
                              sem1).wait()

    fl_in(0)
    fl_in(1)
    for blk in range(NBLK):
        sl = blk % 2
        fl_in_wait()
        if blk >= 2:
            fl_out_wait()

        def convbody(i, _):
            for q in range(CK // 32):
                a, b = plsc.unpack(flb[sl, i, pl.ds(32 * q, 32)],
                                   format=plsc.PackFormat.INTERLEAVED)
                fob[sl, i, pl.ds(32 * q, LANES)] = a
                fob[sl, i, pl.ds(32 * q + LANES, LANES)] = b
            return 0
        lax.fori_loop(0, FB, convbody, 0)
        fl_out(blk)
        if blk + 2 < NBLK:
            fl_in(blk + 2)
    fl_out_wait()
    fl_out_wait()


@functools.cache
def _sc_scatter():
    return functools.partial(
        pl.kernel,
        out_type=jax.ShapeDtypeStruct((G, OUT), jnp.float32),
        mesh=plsc.VectorSubcoreMesh(core_axis_name="c", subcore_axis_name="s"),
        compiler_params=pltpu.CompilerParams(use_tc_tiling_on_sc=False,
                                             needs_layout_passes=False),
        scratch_types=[
            pltpu.VMEM_SHARED((G, CK), jnp.bfloat16),
            pltpu.VMEM((PPT, D), jnp.float32),
            pltpu.VMEM((PPT // PB, PB * D), jnp.int32),
            pltpu.VMEM((PPT // 2, CK), jnp.float32),
            pltpu.VMEM((NSLOT, SROWS, CK), jnp.bfloat16),
            pltpu.VMEM((BINIT, CK), jnp.bfloat16),
            pltpu.VMEM((CK,), jnp.float32),
            pltpu.VMEM((2, FB, CK), jnp.bfloat16),
            pltpu.VMEM((2, FB, CK), jnp.float32),
            pltpu.SemaphoreType.DMA,
            pltpu.SemaphoreType.DMA,
        ],
    )(_sc_body)


def kernel(bev_features, frustum_features, frustum_bev_coordinates,
           last_step_depth_features, W, bias):
    feat_n = frustum_features.reshape(V, C, PV)
    depth_n = last_step_depth_features.reshape(V, D, PV)
    x_n = frustum_bev_coordinates[..., 0].reshape(V, D, PV)
    y_n = frustum_bev_coordinates[..., 1].reshape(V, D, PV)

    fw_c, wpt, rpt = pl.pallas_call(
        _prep_body,
        grid=(V,),
        in_specs=[
            pl.BlockSpec((1, C, PV), lambda v: (v, 0, 0)),
            pl.BlockSpec((C, C), lambda v: (0, 0)),
            pl.BlockSpec((1, D, PV), lambda v: (v, 0, 0)),
            pl.BlockSpec((1, D, PV), lambda v: (v, 0, 0)),
            pl.BlockSpec((1, D, PV), lambda v: (v, 0, 0)),
        ],
        out_specs=[
            pl.BlockSpec((NCHUNK, PV, CK), lambda v: (0, v, 0)),
            pl.BlockSpec((PV, D), lambda v: (v, 0)),
            pl.BlockSpec((PV, D), lambda v: (v, 0)),
        ],
        out_shape=[
            jax.ShapeDtypeStruct((NCHUNK, PIX, CK), jnp.float32),
            jax.ShapeDtypeStruct((PIX, D), jnp.float32),
            jax.ShapeDtypeStruct((PIX, D), jnp.int32),
        ],
    )(feat_n, W, depth_n, x_n, y_n)

    bias_c = bias.reshape(NCHUNK, CK)
    rpt2 = rpt.reshape(PIX // PB, PB * D)

    out = _sc_scatter()(fw_c, wpt, rpt2, bias_c)
    return out.reshape(1, G, OUT)
